# Initial kernel scaffold; baseline (speedup 1.0000x reference)
import functools

import jax
import jax.numpy as jnp
from jax import lax
from jax.experimental import pallas as pl
from jax.experimental.pallas import tpu as pltpu
from jax.experimental.pallas import tpu_sc as plsc

N, E, F_IN, D_E, H, NG, OUT = 10000, 320000, 128, 16, 128, 16, 1
HH = H // 2
_L = 16
_NS = 16
_CH = 128
_NCHUNK = E // _CH
_RPT = N // _NS
_RC = 125


def _lstm_tc(inp, WiT, WgT, bi, bg, blk):
    M, K = inp.shape

    def body(a_ref, wi_ref, wg_ref, bi_ref, bg_ref, o_ref):
        a = a_ref[...]
        gi = jnp.dot(a, wi_ref[...], preferred_element_type=jnp.float32)
        gg = jnp.dot(a, wg_ref[...], preferred_element_type=jnp.float32)
        v = jax.nn.sigmoid(gi + bi_ref[...]) * jnp.tanh(gg + bg_ref[...])
        o_ref[0] = v[:, :HH]
        o_ref[1] = v[:, HH:]

    return pl.pallas_call(
        body,
        grid=(M // blk,),
        in_specs=[
            pl.BlockSpec((blk, K), lambda i: (i, 0)),
            pl.BlockSpec((K, H), lambda i: (0, 0)),
            pl.BlockSpec((K, H), lambda i: (0, 0)),
            pl.BlockSpec((1, H), lambda i: (0, 0)),
            pl.BlockSpec((1, H), lambda i: (0, 0)),
        ],
        out_specs=pl.BlockSpec((2, blk, HH), lambda i: (0, i, 0)),
        out_shape=jax.ShapeDtypeStruct((2, M, HH), jnp.float32),
    )(inp, WiT, WgT, bi, bg)


def _sc_aggregate(x2, ea2, src, dst):
    mesh = plsc.VectorSubcoreMesh(core_axis_name="c", subcore_axis_name="s")

    @functools.partial(
        pl.kernel,
        out_type=jax.ShapeDtypeStruct((2 * N, 2 * HH), jnp.float32),
        mesh=mesh,
        scratch_types=[
            pltpu.VMEM((_CH,), jnp.int32),
            pltpu.VMEM((_CH,), jnp.int32),
            pltpu.VMEM((_CH, HH), jnp.float32),
            pltpu.VMEM((_CH, HH), jnp.float32),
            pltpu.VMEM((_CH, 2 * HH), jnp.float32),
            pltpu.VMEM_SHARED((N, 2 * HH), jnp.float32),
            pltpu.SemaphoreType.DMA,
        ],
    )
    def k(x_hbm, ea_hbm, src_hbm, dst_hbm, out_hbm, src_v, dst_v, xr, er,
          accr, acc_sh, sem):
        c = lax.axis_index("c")
        s = lax.axis_index("s")
        zero = jnp.zeros((_L,), jnp.float32)

        def zrow(r, carry):
            for j in range(2 * HH // _L):
                accr[r, pl.ds(j * _L, _L)] = zero
            return carry

        lax.fori_loop(0, _CH, zrow, None)
        base_r = s * _RPT
        for j in range(_RPT // _RC):
            pltpu.sync_copy(accr.at[pl.ds(0, _RC)],
                            acc_sh.at[pl.ds(base_r + j * _RC, _RC)])
        plsc.subcore_barrier()

        coff = c * N
        nfull = _NCHUNK // _NS
        nch = jnp.where(s < _NCHUNK - nfull * _NS, nfull + 1, nfull)

        def body(i, carry):
            cid = s + i * _NS
            eb = pl.multiple_of(cid * _CH, _CH)
            pltpu.sync_copy(src_hbm.at[pl.ds(eb, _CH)], src_v)
            pltpu.sync_copy(dst_hbm.at[pl.ds(eb, _CH)], dst_v)
            for j in range(_CH // _L):
                src_v[pl.ds(j * _L, _L)] = src_v[pl.ds(j * _L, _L)] + coff
            pltpu.async_copy(x_hbm.at[src_v], xr, sem).wait()
            pltpu.sync_copy(ea_hbm.at[pl.ds(c * E + eb, _CH)], er)

            def crow(r, cc):
                for j in range(HH // _L):
                    xv = xr[r, pl.ds(j * _L, _L)]
                    ev = er[r, pl.ds(j * _L, _L)]
                    msg = jnp.maximum(xv + ev, 0.0) + 1e-7
                    w = jnp.exp(msg)
                    accr[r, pl.ds(j * _L, _L)] = msg * w
                    accr[r, pl.ds(HH + j * _L, _L)] = w
                return cc

            lax.fori_loop(0, _CH, crow, None)
            pltpu.sync_copy(accr, acc_sh.at[dst_v], add=True)
            return carry

        lax.fori_loop(0, nch, body, None)
        plsc.subcore_barrier()
        for j in range(_RPT // _RC):
            r0 = base_r + j * _RC
            pltpu.sync_copy(acc_sh.at[pl.ds(r0, _RC)],
                            out_hbm.at[pl.ds(coff + r0, _RC)])

    return k(x2, ea2, src, dst)


def _conv_tail(num, den, x, oh, w1T, b1, gm, bt, w2T, b2):
    aggr = num / (den + 1e-30)
    out = aggr + x
    h1 = jnp.dot(out, w1T, preferred_element_type=jnp.float32) + b1
    mu = jnp.mean(h1, axis=0, keepdims=True)
    var = jnp.mean((h1 - mu) ** 2, axis=0, keepdims=True)
    h1 = (h1 - mu) / jnp.sqrt(var + 1e-5) * gm + bt
    h1 = jnp.maximum(h1, 0.0)
    h = jnp.maximum(jnp.dot(h1, w2T, preferred_element_type=jnp.float32) + b2,
                    0.0)
    dn = (((0,), (0,)), ((), ()))
    cnt = jnp.maximum(
        lax.dot_general(oh, jnp.ones_like(h[:, :1]), dn,
                        preferred_element_type=jnp.float32), 1.0)
    mean = lax.dot_general(oh, h, dn, preferred_element_type=jnp.float32) / cnt
    var2 = (lax.dot_general(oh, h * h, dn, preferred_element_type=jnp.float32)
            / cnt - mean * mean)
    meanb = jnp.dot(oh, mean, preferred_element_type=jnp.float32)
    varb = jnp.dot(oh, var2, preferred_element_type=jnp.float32)
    return (h - meanb) / jnp.sqrt(varb + 1e-5)


def _onehot(b_ref):
    gid = lax.broadcasted_iota(jnp.int32, (1, NG), 1)
    return (b_ref[...] == gid).astype(jnp.float32)


def _post_conv_tc(num, den, x_in, batch1, w1T, b1, gm, bt, w2T, b2):
    def body(num_ref, den_ref, x_ref, b_ref, w1_ref, b1_ref, gm_ref, bt_ref,
             w2_ref, b2_ref, o_ref):
        hn = _conv_tail(num_ref[...], den_ref[...], x_ref[...], _onehot(b_ref),
                        w1_ref[...], b1_ref[...], gm_ref[...], bt_ref[...],
                        w2_ref[...], b2_ref[...])
        o_ref[0] = hn[:, :HH]
        o_ref[1] = hn[:, HH:]

    return pl.pallas_call(
        body,
        out_shape=jax.ShapeDtypeStruct((2, N, HH), jnp.float32),
    )(num, den, x_in, batch1, w1T, b1, gm, bt, w2T, b2)


def _final_tc(num, den, x_in, batch1, w1T, b1, gm, bt, w2T, b2, wlT, bl):
    def body(num_ref, den_ref, x_ref, b_ref, w1_ref, b1_ref, gm_ref, bt_ref,
             w2_ref, b2_ref, wl_ref, bl_ref, o_ref):
        oh = _onehot(b_ref)
        hn = _conv_tail(num_ref[...], den_ref[...], x_ref[...], oh,
                        w1_ref[...], b1_ref[...], gm_ref[...], bt_ref[...],
                        w2_ref[...], b2_ref[...])
        rows = []
        for g in range(NG):
            mask = b_ref[...] == g
            hm = jnp.where(mask, hn, -3.0e38)
            rows.append(jnp.max(hm, axis=0, keepdims=True))
        pooled = jnp.concatenate(rows, axis=0)
        pooled = jnp.where(pooled <= -1.0e38, 0.0, pooled)
        o_ref[...] = jax.nn.sigmoid(
            jnp.dot(pooled, wl_ref[...], preferred_element_type=jnp.float32)
            + bl_ref[...])

    return pl.pallas_call(
        body,
        out_shape=jax.ShapeDtypeStruct((NG, OUT), jnp.float32),
    )(num, den, x_in, batch1, w1T, b1, gm, bt, w2T, b2, wlT, bl)


def _split_acc(acc):
    num = jnp.concatenate([acc[:N, :HH], acc[N:, :HH]], axis=1)
    den = jnp.concatenate([acc[:N, HH:], acc[N:, HH:]], axis=1)
    return num, den


def kernel(x, edge_index, edge_attr, batch, W_ih_n, W_hh_n, b_ih_n, b_hh_n,
           W_ih_e, W_hh_e, b_ih_e, b_hh_e, W1a, b1a, gma, bta, W2a, b2a,
           W1b, b1b, gmb, btb, W2b, b2b, W_lin, b_lin):
    src, dst = edge_index[0], edge_index[1]
    WiT_n = W_ih_n[0:H].T
    WgT_n = W_ih_n[2 * H:3 * H].T
    bi_n = (b_ih_n[0:H] + b_hh_n[0:H]).reshape(1, H)
    bg_n = (b_ih_n[2 * H:3 * H] + b_hh_n[2 * H:3 * H]).reshape(1, H)
    WiT_e = W_ih_e[0:H].T
    WgT_e = W_ih_e[2 * H:3 * H].T
    bi_e = (b_ih_e[0:H] + b_hh_e[0:H]).reshape(1, H)
    bg_e = (b_ih_e[2 * H:3 * H] + b_hh_e[2 * H:3 * H]).reshape(1, H)

    xh2 = _lstm_tc(x, WiT_n, WgT_n, bi_n, bg_n, 1000)
    ea2 = _lstm_tc(edge_attr, WiT_e, WgT_e, bi_e, bg_e, 4000)
    xh_flat = xh2.reshape(2 * N, HH)
    ea_flat = ea2.reshape(2 * E, HH)
    xh_full = jnp.concatenate([xh2[0], xh2[1]], axis=1)
    batch1 = batch.reshape(N, 1)

    acc = _sc_aggregate(xh_flat, ea_flat, src, dst)
    num, den = _split_acc(acc)
    h2o = _post_conv_tc(num, den, xh_full, batch1,
                        W1a.T, b1a.reshape(1, -1), gma.reshape(1, -1),
                        bta.reshape(1, -1), W2a.T, b2a.reshape(1, -1))
    h_flat = h2o.reshape(2 * N, HH)
    h_full = jnp.concatenate([h2o[0], h2o[1]], axis=1)

    acc2 = _sc_aggregate(h_flat, ea_flat, src, dst)
    num2, den2 = _split_acc(acc2)
    return _final_tc(num2, den2, h_full, batch1,
                     W1b.T, b1b.reshape(1, -1), gmb.reshape(1, -1),
                     btb.reshape(1, -1), W2b.T, b2b.reshape(1, -1),
                     W_lin.T, b_lin.reshape(1, OUT))


# speedup vs baseline: 2.4846x; 2.4846x over previous
import functools

import jax
import jax.numpy as jnp
from jax import lax
from jax.experimental import pallas as pl
from jax.experimental.pallas import tpu as pltpu
from jax.experimental.pallas import tpu_sc as plsc

N, E, F_IN, D_E, H, NG, OUT = 10000, 320000, 128, 16, 128, 16, 1
HH = H // 2
_L = 16
_NS = 16
_CH = 128
_NCHUNK = E // _CH
_RPT = 624


def _lstm_tc(inp, WiT, WgT, bi, bg, blk):
    M, K = inp.shape

    def body(a_ref, wi_ref, wg_ref, bi_ref, bg_ref, o_ref):
        a = a_ref[...]
        gi = jnp.dot(a, wi_ref[...], preferred_element_type=jnp.float32)
        gg = jnp.dot(a, wg_ref[...], preferred_element_type=jnp.float32)
        v = jax.nn.sigmoid(gi + bi_ref[...]) * jnp.tanh(gg + bg_ref[...])
        o_ref[0] = v[:, :HH]
        o_ref[1] = v[:, HH:]

    return pl.pallas_call(
        body,
        grid=(M // blk,),
        in_specs=[
            pl.BlockSpec((blk, K), lambda i: (i, 0)),
            pl.BlockSpec((K, H), lambda i: (0, 0)),
            pl.BlockSpec((K, H), lambda i: (0, 0)),
            pl.BlockSpec((1, H), lambda i: (0, 0)),
            pl.BlockSpec((1, H), lambda i: (0, 0)),
        ],
        out_specs=pl.BlockSpec((2, blk, HH), lambda i: (0, i, 0)),
        out_shape=jax.ShapeDtypeStruct((2, M, HH), jnp.float32),
    )(inp, WiT, WgT, bi, bg)


def _sc_aggregate(x2, ea2, src, dst):
    mesh = plsc.VectorSubcoreMesh(core_axis_name="c", subcore_axis_name="s")

    @functools.partial(
        pl.kernel,
        out_type=jax.ShapeDtypeStruct((2 * N, 2 * HH), jnp.float32),
        mesh=mesh,
        scratch_types=[
            pltpu.VMEM((_CH,), jnp.int32),
            pltpu.VMEM((_CH,), jnp.int32),
            pltpu.VMEM((_CH, HH), jnp.float32),
            pltpu.VMEM((_CH, HH), jnp.float32),
            pltpu.VMEM((_CH, 2 * HH), jnp.float32),
            pltpu.VMEM_SHARED((N, 2 * HH), jnp.float32),
            pltpu.SemaphoreType.DMA,
        ],
        compiler_params=pltpu.CompilerParams(use_tc_tiling_on_sc=False),
    )
    def k(x_hbm, ea_hbm, src_hbm, dst_hbm, out_hbm, src_v, dst_v, xr, er,
          accr, acc_sh, sem):
        c = lax.axis_index("c")
        s = lax.axis_index("s")
        zero = jnp.zeros((_L,), jnp.float32)

        def zrow(r, carry):
            for j in range(2 * HH // _L):
                accr[r, pl.ds(j * _L, _L)] = zero
            return carry

        lax.fori_loop(0, _CH, zrow, None)
        base_r = s * _RPT
        for j, sz in ((0, _CH), (_CH, _CH), (2 * _CH, _CH), (3 * _CH, _CH),
                      (4 * _CH, _RPT - 4 * _CH)):
            pltpu.sync_copy(accr.at[pl.ds(0, sz)],
                            acc_sh.at[pl.ds(base_r + j, sz)])

        @pl.when(s == _NS - 1)
        def _():
            pltpu.sync_copy(accr.at[pl.ds(0, N - _NS * _RPT)],
                            acc_sh.at[pl.ds(_NS * _RPT, N - _NS * _RPT)])

        plsc.subcore_barrier()

        coff = c * N
        nfull = _NCHUNK // _NS
        nch = jnp.where(s < _NCHUNK - nfull * _NS, nfull + 1, nfull)

        def body(i, carry):
            cid = s + i * _NS
            eb = pl.multiple_of(cid * _CH, _CH)
            pltpu.sync_copy(src_hbm.at[pl.ds(eb, _CH)], src_v)
            pltpu.sync_copy(dst_hbm.at[pl.ds(eb, _CH)], dst_v)
            for j in range(_CH // _L):
                src_v[pl.ds(j * _L, _L)] = src_v[pl.ds(j * _L, _L)] + coff
            pltpu.async_copy(x_hbm.at[src_v], xr, sem).wait()
            pltpu.sync_copy(ea_hbm.at[pl.ds(c * E + eb, _CH)], er)

            def crow(r, cc):
                for j in range(HH // _L):
                    xv = xr[r, pl.ds(j * _L, _L)]
                    ev = er[r, pl.ds(j * _L, _L)]
                    msg = jnp.maximum(xv + ev, 0.0) + 1e-7
                    w = jnp.exp(msg)
                    accr[r, pl.ds(j * _L, _L)] = msg * w
                    accr[r, pl.ds(HH + j * _L, _L)] = w
                return cc

            lax.fori_loop(0, _CH, crow, None)
            pltpu.sync_copy(accr, acc_sh.at[dst_v], add=True)
            return carry

        lax.fori_loop(0, nch, body, None)
        plsc.subcore_barrier()
        pltpu.sync_copy(acc_sh.at[pl.ds(base_r, _RPT)],
                        out_hbm.at[pl.ds(coff + base_r, _RPT)])

        @pl.when(s == _NS - 1)
        def _():
            pltpu.sync_copy(acc_sh.at[pl.ds(_NS * _RPT, N - _NS * _RPT)],
                            out_hbm.at[pl.ds(coff + _NS * _RPT,
                                             N - _NS * _RPT)])

    return k(x2, ea2, src, dst)


def _conv_tail(num, den, x, oh, w1T, b1, gm, bt, w2T, b2):
    aggr = num / (den + 1e-30)
    out = aggr + x
    h1 = jnp.dot(out, w1T, preferred_element_type=jnp.float32) + b1
    mu = jnp.mean(h1, axis=0, keepdims=True)
    var = jnp.mean((h1 - mu) ** 2, axis=0, keepdims=True)
    h1 = (h1 - mu) / jnp.sqrt(var + 1e-5) * gm + bt
    h1 = jnp.maximum(h1, 0.0)
    h = jnp.maximum(jnp.dot(h1, w2T, preferred_element_type=jnp.float32) + b2,
                    0.0)
    dn = (((0,), (0,)), ((), ()))
    cnt = jnp.maximum(
        lax.dot_general(oh, jnp.ones_like(h[:, :1]), dn,
                        preferred_element_type=jnp.float32), 1.0)
    mean = lax.dot_general(oh, h, dn, preferred_element_type=jnp.float32) / cnt
    var2 = (lax.dot_general(oh, h * h, dn, preferred_element_type=jnp.float32)
            / cnt - mean * mean)
    meanb = jnp.dot(oh, mean, preferred_element_type=jnp.float32)
    varb = jnp.dot(oh, var2, preferred_element_type=jnp.float32)
    return (h - meanb) / jnp.sqrt(varb + 1e-5)


def _onehot(b_ref):
    gid = lax.broadcasted_iota(jnp.int32, (1, NG), 1)
    return (b_ref[...] == gid).astype(jnp.float32)


def _post_conv_tc(num, den, x_in, batch1, w1T, b1, gm, bt, w2T, b2):
    def body(num_ref, den_ref, x_ref, b_ref, w1_ref, b1_ref, gm_ref, bt_ref,
             w2_ref, b2_ref, o_ref):
        hn = _conv_tail(num_ref[...], den_ref[...], x_ref[...], _onehot(b_ref),
                        w1_ref[...], b1_ref[...], gm_ref[...], bt_ref[...],
                        w2_ref[...], b2_ref[...])
        o_ref[0] = hn[:, :HH]
        o_ref[1] = hn[:, HH:]

    return pl.pallas_call(
        body,
        out_shape=jax.ShapeDtypeStruct((2, N, HH), jnp.float32),
    )(num, den, x_in, batch1, w1T, b1, gm, bt, w2T, b2)


def _final_tc(num, den, x_in, batch1, w1T, b1, gm, bt, w2T, b2, wlT, bl):
    def body(num_ref, den_ref, x_ref, b_ref, w1_ref, b1_ref, gm_ref, bt_ref,
             w2_ref, b2_ref, wl_ref, bl_ref, o_ref):
        oh = _onehot(b_ref)
        hn = _conv_tail(num_ref[...], den_ref[...], x_ref[...], oh,
                        w1_ref[...], b1_ref[...], gm_ref[...], bt_ref[...],
                        w2_ref[...], b2_ref[...])
        rows = []
        for g in range(NG):
            mask = b_ref[...] == g
            hm = jnp.where(mask, hn, -3.0e38)
            rows.append(jnp.max(hm, axis=0, keepdims=True))
        pooled = jnp.concatenate(rows, axis=0)
        pooled = jnp.where(pooled <= -1.0e38, 0.0, pooled)
        o_ref[...] = jax.nn.sigmoid(
            jnp.dot(pooled, wl_ref[...], preferred_element_type=jnp.float32)
            + bl_ref[...])

    return pl.pallas_call(
        body,
        out_shape=jax.ShapeDtypeStruct((NG, OUT), jnp.float32),
    )(num, den, x_in, batch1, w1T, b1, gm, bt, w2T, b2, wlT, bl)


def _split_acc(acc):
    num = jnp.concatenate([acc[:N, :HH], acc[N:, :HH]], axis=1)
    den = jnp.concatenate([acc[:N, HH:], acc[N:, HH:]], axis=1)
    return num, den


def kernel(x, edge_index, edge_attr, batch, W_ih_n, W_hh_n, b_ih_n, b_hh_n,
           W_ih_e, W_hh_e, b_ih_e, b_hh_e, W1a, b1a, gma, bta, W2a, b2a,
           W1b, b1b, gmb, btb, W2b, b2b, W_lin, b_lin):
    src, dst = edge_index[0], edge_index[1]
    WiT_n = W_ih_n[0:H].T
    WgT_n = W_ih_n[2 * H:3 * H].T
    bi_n = (b_ih_n[0:H] + b_hh_n[0:H]).reshape(1, H)
    bg_n = (b_ih_n[2 * H:3 * H] + b_hh_n[2 * H:3 * H]).reshape(1, H)
    WiT_e = W_ih_e[0:H].T
    WgT_e = W_ih_e[2 * H:3 * H].T
    bi_e = (b_ih_e[0:H] + b_hh_e[0:H]).reshape(1, H)
    bg_e = (b_ih_e[2 * H:3 * H] + b_hh_e[2 * H:3 * H]).reshape(1, H)

    xh2 = _lstm_tc(x, WiT_n, WgT_n, bi_n, bg_n, 1000)
    ea2 = _lstm_tc(edge_attr, WiT_e, WgT_e, bi_e, bg_e, 4000)
    xh_flat = xh2.reshape(2 * N, HH)
    ea_flat = ea2.reshape(2 * E, HH)
    xh_full = jnp.concatenate([xh2[0], xh2[1]], axis=1)
    batch1 = batch.reshape(N, 1)

    acc = _sc_aggregate(xh_flat, ea_flat, src, dst)
    num, den = _split_acc(acc)
    h2o = _post_conv_tc(num, den, xh_full, batch1,
                        W1a.T, b1a.reshape(1, -1), gma.reshape(1, -1),
                        bta.reshape(1, -1), W2a.T, b2a.reshape(1, -1))
    h_flat = h2o.reshape(2 * N, HH)
    h_full = jnp.concatenate([h2o[0], h2o[1]], axis=1)

    acc2 = _sc_aggregate(h_flat, ea_flat, src, dst)
    num2, den2 = _split_acc(acc2)
    return _final_tc(num2, den2, h_full, batch1,
                     W1b.T, b1b.reshape(1, -1), gmb.reshape(1, -1),
                     btb.reshape(1, -1), W2b.T, b2b.reshape(1, -1),
                     W_lin.T, b_lin.reshape(1, OUT))


# speedup vs baseline: 4.3004x; 1.7308x over previous
import functools

import jax
import jax.numpy as jnp
from jax import lax
from jax.experimental import pallas as pl
from jax.experimental.pallas import tpu as pltpu
from jax.experimental.pallas import tpu_sc as plsc

N, E, F_IN, D_E, H, NG, OUT = 10000, 320000, 128, 16, 128, 16, 1
HH = H // 2
_L = 16
_NS = 16
_CH = 128
_NCHUNK = E // _CH
_RPT = 624


def _lstm_tc(inp, WiT, WgT, bi, bg, blk):
    M, K = inp.shape

    def body(a_ref, wi_ref, wg_ref, bi_ref, bg_ref, o_ref):
        a = a_ref[...]
        gi = jnp.dot(a, wi_ref[...], preferred_element_type=jnp.float32)
        gg = jnp.dot(a, wg_ref[...], preferred_element_type=jnp.float32)
        v = jax.nn.sigmoid(gi + bi_ref[...]) * jnp.tanh(gg + bg_ref[...])
        o_ref[0] = v[:, :HH]
        o_ref[1] = v[:, HH:]

    return pl.pallas_call(
        body,
        grid=(M // blk,),
        in_specs=[
            pl.BlockSpec((blk, K), lambda i: (i, 0)),
            pl.BlockSpec((K, H), lambda i: (0, 0)),
            pl.BlockSpec((K, H), lambda i: (0, 0)),
            pl.BlockSpec((1, H), lambda i: (0, 0)),
            pl.BlockSpec((1, H), lambda i: (0, 0)),
        ],
        out_specs=pl.BlockSpec((2, blk, HH), lambda i: (0, i, 0)),
        out_shape=jax.ShapeDtypeStruct((2, M, HH), jnp.float32),
    )(inp, WiT, WgT, bi, bg)


def _sc_aggregate(x2, ea2, src, dst):
    mesh = plsc.VectorSubcoreMesh(core_axis_name="c", subcore_axis_name="s")

    @functools.partial(
        pl.kernel,
        out_type=jax.ShapeDtypeStruct((2 * N, 2 * HH), jnp.float32),
        mesh=mesh,
        scratch_types=[
            pltpu.VMEM((_CH,), jnp.int32),
            pltpu.VMEM((_CH,), jnp.int32),
            pltpu.VMEM((_CH, HH), jnp.float32),
            pltpu.VMEM((_CH, HH), jnp.float32),
            pltpu.VMEM((_CH, 2 * HH), jnp.float32),
            pltpu.VMEM_SHARED((N, 2 * HH), jnp.float32),
            pltpu.SemaphoreType.DMA,
        ],
        compiler_params=pltpu.CompilerParams(use_tc_tiling_on_sc=False),
    )
    def k(x_hbm, ea_hbm, src_hbm, dst_hbm, out_hbm, src_v, dst_v, xr, er,
          accr, acc_sh, sem):
        c = lax.axis_index("c")
        s = lax.axis_index("s")
        zero = jnp.zeros((_L,), jnp.float32)

        def zrow(r, carry):
            for j in range(2 * HH // _L):
                accr[r, pl.ds(j * _L, _L)] = zero
            return carry

        lax.fori_loop(0, _CH, zrow, None)
        base_r = s * _RPT
        for j, sz in ((0, _CH), (_CH, _CH), (2 * _CH, _CH), (3 * _CH, _CH),
                      (4 * _CH, _RPT - 4 * _CH)):
            pltpu.sync_copy(accr.at[pl.ds(0, sz)],
                            acc_sh.at[pl.ds(base_r + j, sz)])

        @pl.when(s == _NS - 1)
        def _():
            pltpu.sync_copy(accr.at[pl.ds(0, N - _NS * _RPT)],
                            acc_sh.at[pl.ds(_NS * _RPT, N - _NS * _RPT)])

        plsc.subcore_barrier()

        coff = c * N
        nfull = _NCHUNK // _NS
        nch = jnp.where(s < _NCHUNK - nfull * _NS, nfull + 1, nfull)

        def body(i, carry):
            cid = s + i * _NS
            eb = pl.multiple_of(cid * _CH, _CH)
            pltpu.sync_copy(src_hbm.at[pl.ds(eb, _CH)], src_v)
            pltpu.sync_copy(dst_hbm.at[pl.ds(eb, _CH)], dst_v)
            for j in range(_CH // _L):
                src_v[pl.ds(j * _L, _L)] = src_v[pl.ds(j * _L, _L)] + coff
            pltpu.async_copy(x_hbm.at[src_v], xr, sem).wait()
            pltpu.sync_copy(ea_hbm.at[pl.ds(c * E + eb, _CH)], er)

            @plsc.parallel_loop(0, _CH, step=1, unroll=4)
            def crow(r):
                for j in range(HH // _L):
                    xv = xr[r, pl.ds(j * _L, _L)]
                    ev = er[r, pl.ds(j * _L, _L)]
                    msg = jnp.maximum(xv + ev, 0.0) + 1e-7
                    w = jnp.exp(msg)
                    accr[r, pl.ds(j * _L, _L)] = msg * w
                    accr[r, pl.ds(HH + j * _L, _L)] = w

            pltpu.sync_copy(accr, acc_sh.at[dst_v], add=True)
            return carry

        lax.fori_loop(0, nch, body, None)
        plsc.subcore_barrier()
        pltpu.sync_copy(acc_sh.at[pl.ds(base_r, _RPT)],
                        out_hbm.at[pl.ds(coff + base_r, _RPT)])

        @pl.when(s == _NS - 1)
        def _():
            pltpu.sync_copy(acc_sh.at[pl.ds(_NS * _RPT, N - _NS * _RPT)],
                            out_hbm.at[pl.ds(coff + _NS * _RPT,
                                             N - _NS * _RPT)])

    return k(x2, ea2, src, dst)


def _mlp1_tc(num, den, x_in, w1T, b1):
    def body(num_ref, den_ref, x_ref, w1_ref, b1_ref, o_ref):
        out = num_ref[...] / (den_ref[...] + 1e-30) + x_ref[...]
        o_ref[...] = (jnp.dot(out, w1_ref[...],
                              preferred_element_type=jnp.float32)
                      + b1_ref[...])

    return pl.pallas_call(
        body,
        out_shape=jax.ShapeDtypeStruct((N, 2 * H), jnp.float32),
    )(num, den, x_in, w1T, b1)


def _mlp2_tc(h1, mu, var, gm, bt, w2T, b2):
    def body(h1_ref, mu_ref, var_ref, gm_ref, bt_ref, w2_ref, b2_ref, o_ref):
        h1n = ((h1_ref[...] - mu_ref[...]) / jnp.sqrt(var_ref[...] + 1e-5)
               * gm_ref[...] + bt_ref[...])
        h1n = jnp.maximum(h1n, 0.0)
        o_ref[...] = jnp.maximum(
            jnp.dot(h1n, w2_ref[...], preferred_element_type=jnp.float32)
            + b2_ref[...], 0.0)

    return pl.pallas_call(
        body,
        out_shape=jax.ShapeDtypeStruct((N, H), jnp.float32),
    )(h1, mu, var, gm, bt, w2T, b2)


def _innorm_tc(h, mb, vb):
    def body(h_ref, mb_ref, vb_ref, o_ref):
        hn = (h_ref[...] - mb_ref[...]) / jnp.sqrt(vb_ref[...] + 1e-5)
        o_ref[0] = hn[:, :HH]
        o_ref[1] = hn[:, HH:]

    return pl.pallas_call(
        body,
        out_shape=jax.ShapeDtypeStruct((2, N, HH), jnp.float32),
    )(h, mb, vb)


def _pool_head_tc(h, mb, vb, batch1, wlT, bl):
    def body(h_ref, mb_ref, vb_ref, b_ref, wl_ref, bl_ref, o_ref):
        hn = (h_ref[...] - mb_ref[...]) / jnp.sqrt(vb_ref[...] + 1e-5)
        rows = []
        for g in range(NG):
            mask = b_ref[...] == g
            hm = jnp.where(mask, hn, -3.0e38)
            rows.append(jnp.max(hm, axis=0, keepdims=True))
        pooled = jnp.concatenate(rows, axis=0)
        pooled = jnp.where(pooled <= -1.0e38, 0.0, pooled)
        o_ref[...] = jax.nn.sigmoid(
            jnp.dot(pooled, wl_ref[...], preferred_element_type=jnp.float32)
            + bl_ref[...])

    return pl.pallas_call(
        body,
        out_shape=jax.ShapeDtypeStruct((NG, OUT), jnp.float32),
    )(h, mb, vb, batch1, wlT, bl)


def _in_stats(h, batch):
    ones = jnp.ones((h.shape[0], 1), h.dtype)
    cnt = jnp.maximum(jax.ops.segment_sum(ones, batch, num_segments=NG), 1.0)
    mean = jax.ops.segment_sum(h, batch, num_segments=NG) / cnt
    var = jax.ops.segment_sum(h * h, batch, num_segments=NG) / cnt - mean ** 2
    return mean[batch], var[batch]


def _split_acc(acc):
    num = jnp.concatenate([acc[:N, :HH], acc[N:, :HH]], axis=1)
    den = jnp.concatenate([acc[:N, HH:], acc[N:, HH:]], axis=1)
    return num, den


def kernel(x, edge_index, edge_attr, batch, W_ih_n, W_hh_n, b_ih_n, b_hh_n,
           W_ih_e, W_hh_e, b_ih_e, b_hh_e, W1a, b1a, gma, bta, W2a, b2a,
           W1b, b1b, gmb, btb, W2b, b2b, W_lin, b_lin):
    src, dst = edge_index[0], edge_index[1]
    WiT_n = W_ih_n[0:H].T
    WgT_n = W_ih_n[2 * H:3 * H].T
    bi_n = (b_ih_n[0:H] + b_hh_n[0:H]).reshape(1, H)
    bg_n = (b_ih_n[2 * H:3 * H] + b_hh_n[2 * H:3 * H]).reshape(1, H)
    WiT_e = W_ih_e[0:H].T
    WgT_e = W_ih_e[2 * H:3 * H].T
    bi_e = (b_ih_e[0:H] + b_hh_e[0:H]).reshape(1, H)
    bg_e = (b_ih_e[2 * H:3 * H] + b_hh_e[2 * H:3 * H]).reshape(1, H)

    xh2 = _lstm_tc(x, WiT_n, WgT_n, bi_n, bg_n, 1000)
    ea2 = _lstm_tc(edge_attr, WiT_e, WgT_e, bi_e, bg_e, 4000)
    xh_flat = xh2.reshape(2 * N, HH)
    ea_flat = ea2.reshape(2 * E, HH)
    xh_full = jnp.concatenate([xh2[0], xh2[1]], axis=1)
    batch1 = batch.reshape(N, 1)

    acc = _sc_aggregate(xh_flat, ea_flat, src, dst)
    num, den = _split_acc(acc)
    h1 = _mlp1_tc(num, den, xh_full, W1a.T, b1a.reshape(1, -1))
    mu = h1.mean(axis=0).reshape(1, -1)
    var = h1.var(axis=0).reshape(1, -1)
    h = _mlp2_tc(h1, mu, var, gma.reshape(1, -1), bta.reshape(1, -1),
                 W2a.T, b2a.reshape(1, -1))
    mb, vb = _in_stats(h, batch)
    h2o = _innorm_tc(h, mb, vb)
    h_flat = h2o.reshape(2 * N, HH)
    h_full = jnp.concatenate([h2o[0], h2o[1]], axis=1)

    acc2 = _sc_aggregate(h_flat, ea_flat, src, dst)
    num2, den2 = _split_acc(acc2)
    h1b = _mlp1_tc(num2, den2, h_full, W1b.T, b1b.reshape(1, -1))
    mub = h1b.mean(axis=0).reshape(1, -1)
    varb = h1b.var(axis=0).reshape(1, -1)
    hb = _mlp2_tc(h1b, mub, varb, gmb.reshape(1, -1), btb.reshape(1, -1),
                  W2b.T, b2b.reshape(1, -1))
    mbb, vbb = _in_stats(hb, batch)
    return _pool_head_tc(hb, mbb, vbb, batch1, W_lin.T,
                         b_lin.reshape(1, OUT))


# speedup vs baseline: 6.0402x; 1.4045x over previous
import functools

import jax
import jax.numpy as jnp
from jax import lax
from jax.experimental import pallas as pl
from jax.experimental.pallas import tpu as pltpu
from jax.experimental.pallas import tpu_sc as plsc

N, E, F_IN, D_E, H, NG, OUT = 10000, 320000, 128, 16, 128, 16, 1
HH = H // 2
_L = 16
_NS = 16
_CH = 128
_NCHUNK = E // _CH
_RPT = 624


def _lstm_tc(inp, WiT, WgT, bi, bg, blk):
    M, K = inp.shape

    def body(a_ref, wi_ref, wg_ref, bi_ref, bg_ref, o_ref):
        a = a_ref[...]
        gi = jnp.dot(a, wi_ref[...], preferred_element_type=jnp.float32)
        gg = jnp.dot(a, wg_ref[...], preferred_element_type=jnp.float32)
        v = jax.nn.sigmoid(gi + bi_ref[...]) * jnp.tanh(gg + bg_ref[...])
        o_ref[0] = v[:, :HH]
        o_ref[1] = v[:, HH:]

    return pl.pallas_call(
        body,
        grid=(M // blk,),
        in_specs=[
            pl.BlockSpec((blk, K), lambda i: (i, 0)),
            pl.BlockSpec((K, H), lambda i: (0, 0)),
            pl.BlockSpec((K, H), lambda i: (0, 0)),
            pl.BlockSpec((1, H), lambda i: (0, 0)),
            pl.BlockSpec((1, H), lambda i: (0, 0)),
        ],
        out_specs=pl.BlockSpec((2, blk, HH), lambda i: (0, i, 0)),
        out_shape=jax.ShapeDtypeStruct((2, M, HH), jnp.float32),
    )(inp, WiT, WgT, bi, bg)


def _sc_aggregate(x2, ea2, srcoff, dst):
    mesh = plsc.VectorSubcoreMesh(core_axis_name="c", subcore_axis_name="s")

    @functools.partial(
        pl.kernel,
        out_type=jax.ShapeDtypeStruct((2 * N, 2 * HH), jnp.float32),
        mesh=mesh,
        scratch_types=[
            pltpu.VMEM((_CH,), jnp.int32), pltpu.VMEM((_CH,), jnp.int32),
            pltpu.VMEM((_CH,), jnp.int32), pltpu.VMEM((_CH,), jnp.int32),
            pltpu.VMEM((_CH, HH), jnp.float32),
            pltpu.VMEM((_CH, HH), jnp.float32),
            pltpu.VMEM((_CH, HH), jnp.float32),
            pltpu.VMEM((_CH, HH), jnp.float32),
            pltpu.VMEM((_CH, 2 * HH), jnp.float32),
            pltpu.VMEM_SHARED((N, 2 * HH), jnp.float32),
            pltpu.SemaphoreType.DMA, pltpu.SemaphoreType.DMA,
            pltpu.SemaphoreType.DMA, pltpu.SemaphoreType.DMA,
            pltpu.SemaphoreType.DMA, pltpu.SemaphoreType.DMA,
        ],
        compiler_params=pltpu.CompilerParams(use_tc_tiling_on_sc=False),
    )
    def k(x_hbm, ea_hbm, src_hbm, dst_hbm, out_hbm,
          srcv0, srcv1, dstv0, dstv1, xr0, xr1, er0, er1, accr0,
          acc_sh, is0, is1, ig0, ig1, ie0, ie1):
        c = lax.axis_index("c")
        s = lax.axis_index("s")
        srcv = (srcv0, srcv1)
        dstv = (dstv0, dstv1)
        xr = (xr0, xr1)
        er = (er0, er1)
        accr = (accr0, accr0)
        isem = (is0, is1)
        gsem = (ig0, ig1)
        esem = (ie0, ie1)
        zero = jnp.zeros((_L,), jnp.float32)

        def zrow(r, carry):
            for j in range(2 * HH // _L):
                accr0[r, pl.ds(j * _L, _L)] = zero
            return carry

        lax.fori_loop(0, _CH, zrow, None)
        base_r = s * _RPT
        for j, sz in ((0, _CH), (_CH, _CH), (2 * _CH, _CH), (3 * _CH, _CH),
                      (4 * _CH, _RPT - 4 * _CH)):
            pltpu.sync_copy(accr0.at[pl.ds(0, sz)],
                            acc_sh.at[pl.ds(base_r + j, sz)])

        @pl.when(s == _NS - 1)
        def _():
            pltpu.sync_copy(accr0.at[pl.ds(0, N - _NS * _RPT)],
                            acc_sh.at[pl.ds(_NS * _RPT, N - _NS * _RPT)])

        plsc.subcore_barrier()

        nfull = _NCHUNK // _NS
        nch = jnp.where(s < _NCHUNK - nfull * _NS, nfull + 1, nfull)

        def _eb(kk):
            return pl.multiple_of((s + kk * _NS) * _CH, _CH)

        coff = c * N

        def _issue_idx(kk, b):
            eb = _eb(kk)
            pltpu.async_copy(src_hbm.at[pl.ds(eb, _CH)], srcv[b], isem[b])
            pltpu.async_copy(dst_hbm.at[pl.ds(eb, _CH)], dstv[b], isem[b])

        def _wait_idx(b):
            pltpu.make_async_copy(src_hbm.at[pl.ds(0, _CH)], srcv[b],
                                  isem[b]).wait()
            pltpu.make_async_copy(dst_hbm.at[pl.ds(0, _CH)], dstv[b],
                                  isem[b]).wait()

        def _issue_data(kk, b):
            eb = _eb(kk)
            for j in range(_CH // _L):
                srcv[b][pl.ds(j * _L, _L)] = (srcv[b][pl.ds(j * _L, _L)]
                                              + coff)
            pltpu.async_copy(x_hbm.at[srcv[b]], xr[b], gsem[b])
            pltpu.async_copy(ea_hbm.at[pl.ds(c * E + eb, _CH)], er[b],
                             esem[b])

        def _wait_data(b):
            pltpu.make_async_copy(x_hbm.at[srcv[b]], xr[b], gsem[b]).wait()
            pltpu.make_async_copy(ea_hbm.at[pl.ds(0, _CH)], er[b],
                                  esem[b]).wait()

        _issue_idx(0, 0)
        _wait_idx(0)
        _issue_data(0, 0)
        _issue_idx(1, 1)

        def pair(kp, carry):
            for b in (0, 1):
                kk = 2 * kp + b

                @pl.when(kk < nch)
                def _():
                    @pl.when(kk + 1 < nch)
                    def _():
                        _wait_idx(1 - b)
                        _issue_data(kk + 1, 1 - b)

                    _wait_data(b)

                    @plsc.parallel_loop(0, _CH, step=1, unroll=4)
                    def crow(r):
                        for j in range(HH // _L):
                            xv = xr[b][r, pl.ds(j * _L, _L)]
                            ev = er[b][r, pl.ds(j * _L, _L)]
                            msg = jnp.maximum(xv + ev, 0.0) + 1e-7
                            w = jnp.exp(msg)
                            accr[b][r, pl.ds(j * _L, _L)] = msg * w
                            accr[b][r, pl.ds(HH + j * _L, _L)] = w

                    pltpu.sync_copy(accr[b], acc_sh.at[dstv[b]], add=True)

                    @pl.when(kk + 2 < nch)
                    def _():
                        _issue_idx(kk + 2, b)

            return carry

        lax.fori_loop(0, (nfull + 2) // 2, pair, None)
        plsc.subcore_barrier()
        pltpu.sync_copy(acc_sh.at[pl.ds(base_r, _RPT)],
                        out_hbm.at[pl.ds(coff + base_r, _RPT)])

        @pl.when(s == _NS - 1)
        def _():
            pltpu.sync_copy(acc_sh.at[pl.ds(_NS * _RPT, N - _NS * _RPT)],
                            out_hbm.at[pl.ds(coff + _NS * _RPT,
                                             N - _NS * _RPT)])

    return k(x2, ea2, srcoff, dst)


def _mlp1_tc(num, den, x_in, w1T, b1):
    def body(num_ref, den_ref, x_ref, w1_ref, b1_ref, o_ref):
        out = num_ref[...] / (den_ref[...] + 1e-30) + x_ref[...]
        o_ref[...] = (jnp.dot(out, w1_ref[...],
                              preferred_element_type=jnp.float32)
                      + b1_ref[...])

    return pl.pallas_call(
        body,
        out_shape=jax.ShapeDtypeStruct((N, 2 * H), jnp.float32),
    )(num, den, x_in, w1T, b1)


def _mlp2_tc(h1, mu, var, gm, bt, w2T, b2):
    def body(h1_ref, mu_ref, var_ref, gm_ref, bt_ref, w2_ref, b2_ref, o_ref):
        h1n = ((h1_ref[...] - mu_ref[...]) / jnp.sqrt(var_ref[...] + 1e-5)
               * gm_ref[...] + bt_ref[...])
        h1n = jnp.maximum(h1n, 0.0)
        o_ref[...] = jnp.maximum(
            jnp.dot(h1n, w2_ref[...], preferred_element_type=jnp.float32)
            + b2_ref[...], 0.0)

    return pl.pallas_call(
        body,
        out_shape=jax.ShapeDtypeStruct((N, H), jnp.float32),
    )(h1, mu, var, gm, bt, w2T, b2)


def _innorm_tc(h, mb, vb):
    def body(h_ref, mb_ref, vb_ref, o_ref):
        hn = (h_ref[...] - mb_ref[...]) / jnp.sqrt(vb_ref[...] + 1e-5)
        o_ref[0] = hn[:, :HH]
        o_ref[1] = hn[:, HH:]

    return pl.pallas_call(
        body,
        out_shape=jax.ShapeDtypeStruct((2, N, HH), jnp.float32),
    )(h, mb, vb)


def _pool_head_tc(h, mb, vb, batch1, wlT, bl):
    def body(h_ref, mb_ref, vb_ref, b_ref, wl_ref, bl_ref, o_ref):
        hn = (h_ref[...] - mb_ref[...]) / jnp.sqrt(vb_ref[...] + 1e-5)
        rows = []
        for g in range(NG):
            mask = b_ref[...] == g
            hm = jnp.where(mask, hn, -3.0e38)
            rows.append(jnp.max(hm, axis=0, keepdims=True))
        pooled = jnp.concatenate(rows, axis=0)
        pooled = jnp.where(pooled <= -1.0e38, 0.0, pooled)
        o_ref[...] = jax.nn.sigmoid(
            jnp.dot(pooled, wl_ref[...], preferred_element_type=jnp.float32)
            + bl_ref[...])

    return pl.pallas_call(
        body,
        out_shape=jax.ShapeDtypeStruct((NG, OUT), jnp.float32),
    )(h, mb, vb, batch1, wlT, bl)


def _in_stats(h, batch):
    ones = jnp.ones((h.shape[0], 1), h.dtype)
    cnt = jnp.maximum(jax.ops.segment_sum(ones, batch, num_segments=NG), 1.0)
    mean = jax.ops.segment_sum(h, batch, num_segments=NG) / cnt
    var = jax.ops.segment_sum(h * h, batch, num_segments=NG) / cnt - mean ** 2
    return mean[batch], var[batch]


def _split_acc(acc):
    num = jnp.concatenate([acc[:N, :HH], acc[N:, :HH]], axis=1)
    den = jnp.concatenate([acc[:N, HH:], acc[N:, HH:]], axis=1)
    return num, den


def kernel(x, edge_index, edge_attr, batch, W_ih_n, W_hh_n, b_ih_n, b_hh_n,
           W_ih_e, W_hh_e, b_ih_e, b_hh_e, W1a, b1a, gma, bta, W2a, b2a,
           W1b, b1b, gmb, btb, W2b, b2b, W_lin, b_lin):
    src, dst = edge_index[0], edge_index[1]
    WiT_n = W_ih_n[0:H].T
    WgT_n = W_ih_n[2 * H:3 * H].T
    bi_n = (b_ih_n[0:H] + b_hh_n[0:H]).reshape(1, H)
    bg_n = (b_ih_n[2 * H:3 * H] + b_hh_n[2 * H:3 * H]).reshape(1, H)
    WiT_e = W_ih_e[0:H].T
    WgT_e = W_ih_e[2 * H:3 * H].T
    bi_e = (b_ih_e[0:H] + b_hh_e[0:H]).reshape(1, H)
    bg_e = (b_ih_e[2 * H:3 * H] + b_hh_e[2 * H:3 * H]).reshape(1, H)

    xh2 = _lstm_tc(x, WiT_n, WgT_n, bi_n, bg_n, 1000)
    ea2 = _lstm_tc(edge_attr, WiT_e, WgT_e, bi_e, bg_e, 4000)
    xh_flat = xh2.reshape(2 * N, HH)
    ea_flat = ea2.reshape(2 * E, HH)
    xh_full = jnp.concatenate([xh2[0], xh2[1]], axis=1)
    batch1 = batch.reshape(N, 1)
    acc = _sc_aggregate(xh_flat, ea_flat, src, dst)
    num, den = _split_acc(acc)
    h1 = _mlp1_tc(num, den, xh_full, W1a.T, b1a.reshape(1, -1))
    mu = h1.mean(axis=0).reshape(1, -1)
    var = h1.var(axis=0).reshape(1, -1)
    h = _mlp2_tc(h1, mu, var, gma.reshape(1, -1), bta.reshape(1, -1),
                 W2a.T, b2a.reshape(1, -1))
    mb, vb = _in_stats(h, batch)
    h2o = _innorm_tc(h, mb, vb)
    h_flat = h2o.reshape(2 * N, HH)
    h_full = jnp.concatenate([h2o[0], h2o[1]], axis=1)

    acc2 = _sc_aggregate(h_flat, ea_flat, src, dst)
    num2, den2 = _split_acc(acc2)
    h1b = _mlp1_tc(num2, den2, h_full, W1b.T, b1b.reshape(1, -1))
    mub = h1b.mean(axis=0).reshape(1, -1)
    varb = h1b.var(axis=0).reshape(1, -1)
    hb = _mlp2_tc(h1b, mub, varb, gmb.reshape(1, -1), btb.reshape(1, -1),
                  W2b.T, b2b.reshape(1, -1))
    mbb, vbb = _in_stats(hb, batch)
    return _pool_head_tc(hb, mbb, vbb, batch1, W_lin.T,
                         b_lin.reshape(1, OUT))


# speedup vs baseline: 6.0800x; 1.0066x over previous
import functools

import jax
import jax.numpy as jnp
from jax import lax
from jax.experimental import pallas as pl
from jax.experimental.pallas import tpu as pltpu
from jax.experimental.pallas import tpu_sc as plsc

N, E, F_IN, D_E, H, NG, OUT = 10000, 320000, 128, 16, 128, 16, 1
HH = H // 2
_L = 16
_NS = 16
_CH = 128
_NCHUNK = E // _CH
_RPT = 624


def _lstm_tc(inp, WiT, WgT, bi, bg, blk):
    M, K = inp.shape

    def body(a_ref, wi_ref, wg_ref, bi_ref, bg_ref, o_ref):
        a = a_ref[...]
        gi = jnp.dot(a, wi_ref[...], preferred_element_type=jnp.float32)
        gg = jnp.dot(a, wg_ref[...], preferred_element_type=jnp.float32)
        v = jax.nn.sigmoid(gi + bi_ref[...]) * jnp.tanh(gg + bg_ref[...])
        o_ref[0] = v[:, :HH]
        o_ref[1] = v[:, HH:]

    return pl.pallas_call(
        body,
        grid=(M // blk,),
        in_specs=[
            pl.BlockSpec((blk, K), lambda i: (i, 0)),
            pl.BlockSpec((K, H), lambda i: (0, 0)),
            pl.BlockSpec((K, H), lambda i: (0, 0)),
            pl.BlockSpec((1, H), lambda i: (0, 0)),
            pl.BlockSpec((1, H), lambda i: (0, 0)),
        ],
        out_specs=pl.BlockSpec((2, blk, HH), lambda i: (0, i, 0)),
        out_shape=jax.ShapeDtypeStruct((2, M, HH), jnp.float32),
    )(inp, WiT, WgT, bi, bg)


def _sc_aggregate(x2, ea2, srcoff, dst):
    mesh = plsc.VectorSubcoreMesh(core_axis_name="c", subcore_axis_name="s")

    @functools.partial(
        pl.kernel,
        out_type=jax.ShapeDtypeStruct((2 * N, 2 * HH), jnp.float32),
        mesh=mesh,
        scratch_types=[
            pltpu.VMEM((_CH,), jnp.int32), pltpu.VMEM((_CH,), jnp.int32),
            pltpu.VMEM((_CH,), jnp.int32), pltpu.VMEM((_CH,), jnp.int32),
            pltpu.VMEM((_CH, HH), jnp.float32),
            pltpu.VMEM((_CH, HH), jnp.float32),
            pltpu.VMEM((_CH, HH), jnp.float32),
            pltpu.VMEM((_CH, HH), jnp.float32),
            pltpu.VMEM((_CH, 2 * HH), jnp.float32),
            pltpu.VMEM_SHARED((N, 2 * HH), jnp.float32),
            pltpu.SemaphoreType.DMA, pltpu.SemaphoreType.DMA,
            pltpu.SemaphoreType.DMA, pltpu.SemaphoreType.DMA,
            pltpu.SemaphoreType.DMA, pltpu.SemaphoreType.DMA,
        ],
        compiler_params=pltpu.CompilerParams(use_tc_tiling_on_sc=False),
    )
    def k(x_hbm, ea_hbm, src_hbm, dst_hbm, out_hbm,
          srcv0, srcv1, dstv0, dstv1, xr0, xr1, er0, er1, accr0,
          acc_sh, is0, is1, ig0, ig1, ie0, ie1):
        c = lax.axis_index("c")
        s = lax.axis_index("s")
        srcv = (srcv0, srcv1)
        dstv = (dstv0, dstv1)
        xr = (xr0, xr1)
        er = (er0, er1)
        accr = (accr0, accr0)
        isem = (is0, is1)
        gsem = (ig0, ig1)
        esem = (ie0, ie1)
        zero = jnp.zeros((_L,), jnp.float32)

        def zrow(r, carry):
            for j in range(2 * HH // _L):
                accr0[r, pl.ds(j * _L, _L)] = zero
            return carry

        lax.fori_loop(0, _CH, zrow, None)
        base_r = s * _RPT
        for j, sz in ((0, _CH), (_CH, _CH), (2 * _CH, _CH), (3 * _CH, _CH),
                      (4 * _CH, _RPT - 4 * _CH)):
            pltpu.sync_copy(accr0.at[pl.ds(0, sz)],
                            acc_sh.at[pl.ds(base_r + j, sz)])

        @pl.when(s == _NS - 1)
        def _():
            pltpu.sync_copy(accr0.at[pl.ds(0, N - _NS * _RPT)],
                            acc_sh.at[pl.ds(_NS * _RPT, N - _NS * _RPT)])

        plsc.subcore_barrier()

        nfull = _NCHUNK // _NS
        nch = jnp.where(s < _NCHUNK - nfull * _NS, nfull + 1, nfull)

        def _eb(kk):
            return pl.multiple_of((s + kk * _NS) * _CH, _CH)

        coff = c * N

        def _issue_idx(kk, b):
            eb = _eb(kk)
            pltpu.async_copy(src_hbm.at[pl.ds(eb, _CH)], srcv[b], isem[b])
            pltpu.async_copy(dst_hbm.at[pl.ds(eb, _CH)], dstv[b], isem[b])

        def _wait_idx(b):
            pltpu.make_async_copy(src_hbm.at[pl.ds(0, _CH)], srcv[b],
                                  isem[b]).wait()
            pltpu.make_async_copy(dst_hbm.at[pl.ds(0, _CH)], dstv[b],
                                  isem[b]).wait()

        def _issue_data(kk, b):
            eb = _eb(kk)
            for j in range(_CH // _L):
                srcv[b][pl.ds(j * _L, _L)] = (srcv[b][pl.ds(j * _L, _L)]
                                              + coff)
            pltpu.async_copy(x_hbm.at[srcv[b]], xr[b], gsem[b])
            pltpu.async_copy(ea_hbm.at[pl.ds(c * E + eb, _CH)], er[b],
                             esem[b])

        def _wait_data(b):
            pltpu.make_async_copy(x_hbm.at[srcv[b]], xr[b], gsem[b]).wait()
            pltpu.make_async_copy(ea_hbm.at[pl.ds(0, _CH)], er[b],
                                  esem[b]).wait()

        _issue_idx(0, 0)
        _wait_idx(0)
        _issue_data(0, 0)
        _issue_idx(1, 1)

        def pair(kp, carry):
            for b in (0, 1):
                kk = 2 * kp + b

                @pl.when(kk < nch)
                def _():
                    @pl.when(kk + 1 < nch)
                    def _():
                        _wait_idx(1 - b)
                        _issue_data(kk + 1, 1 - b)

                    _wait_data(b)

                    @plsc.parallel_loop(0, _CH, step=1, unroll=8)
                    def crow(r):
                        for j in range(HH // _L):
                            xv = xr[b][r, pl.ds(j * _L, _L)]
                            ev = er[b][r, pl.ds(j * _L, _L)]
                            msg = jnp.maximum(xv + ev, 0.0) + 1e-7
                            w = jnp.exp(msg)
                            accr[b][r, pl.ds(j * _L, _L)] = msg * w
                            accr[b][r, pl.ds(HH + j * _L, _L)] = w

                    pltpu.sync_copy(accr[b], acc_sh.at[dstv[b]], add=True)

                    @pl.when(kk + 2 < nch)
                    def _():
                        _issue_idx(kk + 2, b)

            return carry

        lax.fori_loop(0, (nfull + 2) // 2, pair, None)
        plsc.subcore_barrier()
        pltpu.sync_copy(acc_sh.at[pl.ds(base_r, _RPT)],
                        out_hbm.at[pl.ds(coff + base_r, _RPT)])

        @pl.when(s == _NS - 1)
        def _():
            pltpu.sync_copy(acc_sh.at[pl.ds(_NS * _RPT, N - _NS * _RPT)],
                            out_hbm.at[pl.ds(coff + _NS * _RPT,
                                             N - _NS * _RPT)])

    return k(x2, ea2, srcoff, dst)


def _mlp1_tc(acc, x2, w1T, b1):
    def body(acc_ref, x_ref, w1_ref, b1_ref, o_ref):
        a = acc_ref[...]
        num = jnp.concatenate([a[:N, :HH], a[N:, :HH]], axis=1)
        den = jnp.concatenate([a[:N, HH:], a[N:, HH:]], axis=1)
        x = jnp.concatenate([x_ref[0], x_ref[1]], axis=1)
        out = num / (den + 1e-30) + x
        o_ref[...] = (jnp.dot(out, w1_ref[...],
                              preferred_element_type=jnp.float32)
                      + b1_ref[...])

    return pl.pallas_call(
        body,
        out_shape=jax.ShapeDtypeStruct((N, 2 * H), jnp.float32),
    )(acc, x2, w1T, b1)


def _mlp2_tc(h1, mu, var, gm, bt, w2T, b2):
    def body(h1_ref, mu_ref, var_ref, gm_ref, bt_ref, w2_ref, b2_ref, o_ref):
        h1n = ((h1_ref[...] - mu_ref[...]) / jnp.sqrt(var_ref[...] + 1e-5)
               * gm_ref[...] + bt_ref[...])
        h1n = jnp.maximum(h1n, 0.0)
        o_ref[...] = jnp.maximum(
            jnp.dot(h1n, w2_ref[...], preferred_element_type=jnp.float32)
            + b2_ref[...], 0.0)

    return pl.pallas_call(
        body,
        out_shape=jax.ShapeDtypeStruct((N, H), jnp.float32),
    )(h1, mu, var, gm, bt, w2T, b2)


def _innorm_tc(h, mb, vb):
    def body(h_ref, mb_ref, vb_ref, o_ref):
        hn = (h_ref[...] - mb_ref[...]) / jnp.sqrt(vb_ref[...] + 1e-5)
        o_ref[0] = hn[:, :HH]
        o_ref[1] = hn[:, HH:]

    return pl.pallas_call(
        body,
        out_shape=jax.ShapeDtypeStruct((2, N, HH), jnp.float32),
    )(h, mb, vb)


def _pool_head_tc(h, mb, vb, batch1, wlT, bl):
    def body(h_ref, mb_ref, vb_ref, b_ref, wl_ref, bl_ref, o_ref):
        hn = (h_ref[...] - mb_ref[...]) / jnp.sqrt(vb_ref[...] + 1e-5)
        rows = []
        for g in range(NG):
            mask = b_ref[...] == g
            hm = jnp.where(mask, hn, -3.0e38)
            rows.append(jnp.max(hm, axis=0, keepdims=True))
        pooled = jnp.concatenate(rows, axis=0)
        pooled = jnp.where(pooled <= -1.0e38, 0.0, pooled)
        o_ref[...] = jax.nn.sigmoid(
            jnp.dot(pooled, wl_ref[...], preferred_element_type=jnp.float32)
            + bl_ref[...])

    return pl.pallas_call(
        body,
        out_shape=jax.ShapeDtypeStruct((NG, OUT), jnp.float32),
    )(h, mb, vb, batch1, wlT, bl)


def _in_stats(h, batch):
    ones = jnp.ones((h.shape[0], 1), h.dtype)
    cnt = jnp.maximum(jax.ops.segment_sum(ones, batch, num_segments=NG), 1.0)
    mean = jax.ops.segment_sum(h, batch, num_segments=NG) / cnt
    var = jax.ops.segment_sum(h * h, batch, num_segments=NG) / cnt - mean ** 2
    return mean[batch], var[batch]


def kernel(x, edge_index, edge_attr, batch, W_ih_n, W_hh_n, b_ih_n, b_hh_n,
           W_ih_e, W_hh_e, b_ih_e, b_hh_e, W1a, b1a, gma, bta, W2a, b2a,
           W1b, b1b, gmb, btb, W2b, b2b, W_lin, b_lin):
    src, dst = edge_index[0], edge_index[1]
    WiT_n = W_ih_n[0:H].T
    WgT_n = W_ih_n[2 * H:3 * H].T
    bi_n = (b_ih_n[0:H] + b_hh_n[0:H]).reshape(1, H)
    bg_n = (b_ih_n[2 * H:3 * H] + b_hh_n[2 * H:3 * H]).reshape(1, H)
    WiT_e = W_ih_e[0:H].T
    WgT_e = W_ih_e[2 * H:3 * H].T
    bi_e = (b_ih_e[0:H] + b_hh_e[0:H]).reshape(1, H)
    bg_e = (b_ih_e[2 * H:3 * H] + b_hh_e[2 * H:3 * H]).reshape(1, H)

    xh2 = _lstm_tc(x, WiT_n, WgT_n, bi_n, bg_n, 1000)
    ea2 = _lstm_tc(edge_attr, WiT_e, WgT_e, bi_e, bg_e, 4000)
    xh_flat = xh2.reshape(2 * N, HH)
    ea_flat = ea2.reshape(2 * E, HH)
    batch1 = batch.reshape(N, 1)
    acc = _sc_aggregate(xh_flat, ea_flat, src, dst)
    h1 = _mlp1_tc(acc, xh2, W1a.T, b1a.reshape(1, -1))
    mu = h1.mean(axis=0).reshape(1, -1)
    var = h1.var(axis=0).reshape(1, -1)
    h = _mlp2_tc(h1, mu, var, gma.reshape(1, -1), bta.reshape(1, -1),
                 W2a.T, b2a.reshape(1, -1))
    mb, vb = _in_stats(h, batch)
    h2o = _innorm_tc(h, mb, vb)
    h_flat = h2o.reshape(2 * N, HH)

    acc2 = _sc_aggregate(h_flat, ea_flat, src, dst)
    h1b = _mlp1_tc(acc2, h2o, W1b.T, b1b.reshape(1, -1))
    mub = h1b.mean(axis=0).reshape(1, -1)
    varb = h1b.var(axis=0).reshape(1, -1)
    hb = _mlp2_tc(h1b, mub, varb, gmb.reshape(1, -1), btb.reshape(1, -1),
                  W2b.T, b2b.reshape(1, -1))
    mbb, vbb = _in_stats(hb, batch)
    return _pool_head_tc(hb, mbb, vbb, batch1, W_lin.T,
                         b_lin.reshape(1, OUT))
